# Initial kernel scaffold; baseline (speedup 1.0000x reference)
import jax
import jax.numpy as jnp
from jax.experimental import pallas as pl


def kernel(edge_index, feat_src, feat_dst, edge_weight, W_n, W_e, Q, K, att_w, att_b, out_w, out_b):
    raise NotImplementedError("write your pallas kernel here")


# speedup vs baseline: 5.3382x; 5.3382x over previous
import functools

import jax
import jax.numpy as jnp
from jax import lax
from jax.experimental import pallas as pl
from jax.experimental.pallas import tpu as pltpu
from jax.experimental.pallas import tpu_sc as plsc

N = 10000
E = 320000
D_IN = 128
D_OUT = 128
D_EDGE = 16
DH = D_OUT // 2

NC = 2
NS = 16
EPT = E // NS
CHUNK = 80
NCHUNK = EPT // CHUNK
NROWBLK = N // CHUNK
NBLK = 1000
SW_COLS = 32


def _node_proj_body(fs_ref, fd_ref, wn_ref, q_ref, kw_ref, a1_ref, a2_ref,
                    hd_ref, klo_ref, khi_ref, ad_ref, as_ref):
    f32 = jnp.float32
    hd = jnp.dot(fd_ref[...], wn_ref[...], preferred_element_type=f32)
    hs = jnp.dot(fs_ref[...], wn_ref[...], preferred_element_type=f32)
    kk = jnp.dot(hs, kw_ref[...], preferred_element_type=f32)
    hd_ref[...] = hd
    klo_ref[...] = kk[:, 0:DH]
    khi_ref[...] = kk[:, DH:D_OUT]
    qa1 = jnp.dot(q_ref[...], a1_ref[...], preferred_element_type=f32)
    ad_ref[...] = jnp.dot(hd, qa1, preferred_element_type=f32)
    as_ref[...] = jnp.dot(kk, a2_ref[...], preferred_element_type=f32)


def _node_proj(feat_src, feat_dst, W_n, Q, K, a1, a2):
    nblocks = N // NBLK
    full = lambda shape: pl.BlockSpec(shape, lambda i: (0, 0))
    blk = lambda w: pl.BlockSpec((NBLK, w), lambda i: (i, 0))
    return pl.pallas_call(
        _node_proj_body,
        grid=(nblocks,),
        in_specs=[
            blk(D_IN), blk(D_IN),
            full((D_IN, D_OUT)), full((D_OUT, D_OUT)), full((D_OUT, D_OUT)),
            full((D_OUT, 1)), full((D_OUT, 1)),
        ],
        out_specs=[blk(D_OUT), blk(DH), blk(DH), blk(1), blk(1)],
        out_shape=[
            jax.ShapeDtypeStruct((N, D_OUT), jnp.float32),
            jax.ShapeDtypeStruct((N, DH), jnp.float32),
            jax.ShapeDtypeStruct((N, DH), jnp.float32),
            jax.ShapeDtypeStruct((N, 1), jnp.float32),
            jax.ShapeDtypeStruct((N, 1), jnp.float32),
        ],
    )(feat_src, feat_dst, W_n, Q, K, a1, a2)


def _beta_body(ew_ref, we_ref, a3_ref, ab_ref, beta_ref):
    f32 = jnp.float32
    wv = jnp.dot(we_ref[...], a3_ref[...], preferred_element_type=f32)
    beta_ref[...] = (
        jnp.dot(ew_ref[...], wv, preferred_element_type=f32) + ab_ref[0, 0]
    )


def _beta(edge_weight, W_e, a3, ab):
    eb = 8000
    return pl.pallas_call(
        _beta_body,
        grid=(E // eb,),
        in_specs=[
            pl.BlockSpec((eb, D_EDGE), lambda i: (i, 0)),
            pl.BlockSpec((D_EDGE, D_OUT), lambda i: (0, 0)),
            pl.BlockSpec((D_OUT, 1), lambda i: (0, 0)),
            pl.BlockSpec((1, 1), lambda i: (0, 0)),
        ],
        out_specs=pl.BlockSpec((eb, 1), lambda i: (i, 0)),
        out_shape=jax.ShapeDtypeStruct((E, 1), jnp.float32),
    )(edge_weight, W_e, a3, ab)


def _sc_edge_body(src_hbm, dst_hbm, beta_hbm, ad_hbm, as_hbm, klo_hbm,
                  khi_hbm, ew_hbm,
                  sk_out, sw_out,
                  ad_v, as_v, src_v, dst_v, beta_v, ex_v, ew_v, ews_v,
                  krows_v, sk_sh, sw_sh, sem):
    c = lax.axis_index("c")
    s = lax.axis_index("s")
    z16 = jnp.zeros((16,), jnp.float32)
    lane = lax.broadcasted_iota(jnp.int32, (16,), 0)

    pltpu.sync_copy(ad_hbm, ad_v)
    pltpu.sync_copy(as_hbm, as_v)

    def zero_body(j, _):
        for v in range(DH // 16):
            krows_v[j, pl.ds(v * 16, 16)] = z16
        for v in range(SW_COLS // 16):
            ews_v[j, pl.ds(v * 16, 16)] = z16
        return 0

    lax.fori_loop(0, CHUNK, zero_body, 0)
    for t in range(pl.cdiv(NROWBLK, NS)):
        bid = s + NS * t
        @pl.when(bid < NROWBLK)
        def _():
            r0 = bid * CHUNK
            pltpu.sync_copy(krows_v, sk_sh.at[pl.ds(r0, CHUNK)])
            pltpu.sync_copy(ews_v, sw_sh.at[pl.ds(r0, CHUNK)])
    plsc.subcore_barrier()

    def chunk_body(i, _):
        base = s * EPT + i * CHUNK
        pltpu.sync_copy(src_hbm.at[pl.ds(base, CHUNK)], src_v)
        pltpu.sync_copy(dst_hbm.at[pl.ds(base, CHUNK)], dst_v)
        pltpu.sync_copy(beta_hbm.at[pl.ds(base, CHUNK)], beta_v)

        @pl.when(c == 0)
        def _():
            pltpu.async_copy(klo_hbm.at[src_v], krows_v, sem).wait()

        @pl.when(c == 1)
        def _():
            pltpu.async_copy(khi_hbm.at[src_v], krows_v, sem).wait()

        for g in range(CHUNK // 16):
            di = dst_v[pl.ds(g * 16, 16)]
            si = src_v[pl.ds(g * 16, 16)]
            z = (plsc.load_gather(ad_v, [di]) + plsc.load_gather(as_v, [si])
                 + beta_v[pl.ds(g * 16, 16)])
            lg = jnp.where(z >= 0.0, z, 0.2 * z)
            ex_v[pl.ds(g * 16, 16)] = jnp.exp(lg)

        def row_body(j, _):
            bex = plsc.load_gather(ex_v, [jnp.zeros((16,), jnp.int32) + j])
            for v in range(DH // 16):
                krows_v[j, pl.ds(v * 16, 16)] = (
                    krows_v[j, pl.ds(v * 16, 16)] * bex)
            return 0

        lax.fori_loop(0, CHUNK, row_body, 0)
        pltpu.sync_copy(krows_v, sk_sh.at[dst_v], add=True)

        do_sw = jnp.logical_or(
            jnp.logical_and(c == 0, i < NCHUNK // 2),
            jnp.logical_and(c == 1, i >= NCHUNK // 2))

        @pl.when(do_sw)
        def _():
            pltpu.sync_copy(ew_hbm.at[pl.ds(base, CHUNK)], ew_v)

            def ew_body(j, _):
                bex = plsc.load_gather(
                    ex_v, [jnp.zeros((16,), jnp.int32) + j])
                ews_v[j, pl.ds(0, 16)] = ew_v[j, pl.ds(0, 16)] * bex
                ews_v[j, pl.ds(16, 16)] = jnp.where(lane == 0, bex, 0.0)
                return 0

            lax.fori_loop(0, CHUNK, ew_body, 0)
            pltpu.sync_copy(ews_v, sw_sh.at[dst_v], add=True)

        return 0

    lax.fori_loop(0, NCHUNK, chunk_body, 0)
    plsc.subcore_barrier()

    for t in range(pl.cdiv(NROWBLK, NS)):
        bid = s + NS * t
        @pl.when(bid < NROWBLK)
        def _():
            r0 = bid * CHUNK
            pltpu.sync_copy(sk_sh.at[pl.ds(r0, CHUNK)],
                            sk_out.at[c, pl.ds(r0, CHUNK)])
            pltpu.sync_copy(sw_sh.at[pl.ds(r0, CHUNK)],
                            sw_out.at[c, pl.ds(r0, CHUNK)])


def _sc_edge(src, dst, beta, ad, asrc, klo, khi, edge_weight):
    mesh = plsc.VectorSubcoreMesh(
        core_axis_name="c", subcore_axis_name="s",
        num_cores=NC, num_subcores=NS)
    f32 = jnp.float32
    return pl.kernel(
        _sc_edge_body,
        out_type=[
            jax.ShapeDtypeStruct((NC, N, DH), f32),
            jax.ShapeDtypeStruct((NC, N, SW_COLS), f32),
        ],
        mesh=mesh,
        compiler_params=pltpu.CompilerParams(
            needs_layout_passes=False, use_tc_tiling_on_sc=False),
        scratch_types=[
            pltpu.VMEM((N,), f32),
            pltpu.VMEM((N,), f32),
            pltpu.VMEM((CHUNK,), jnp.int32),
            pltpu.VMEM((CHUNK,), jnp.int32),
            pltpu.VMEM((CHUNK,), f32),
            pltpu.VMEM((CHUNK,), f32),
            pltpu.VMEM((CHUNK, D_EDGE), f32),
            pltpu.VMEM((CHUNK, SW_COLS), f32),
            pltpu.VMEM((CHUNK, DH), f32),
            pltpu.VMEM_SHARED((N, DH), f32),
            pltpu.VMEM_SHARED((N, SW_COLS), f32),
            pltpu.SemaphoreType.DMA,
        ],
    )(src, dst, beta, ad, asrc, klo, khi, edge_weight)


def _final_body(hd_ref, skp_ref, swp_ref, we_ref, ow_ref, ob_ref, out_ref):
    f32 = jnp.float32
    sk = jnp.concatenate([skp_ref[0], skp_ref[1]], axis=1)
    sw = swp_ref[0] + swp_ref[1]
    denom = sw[:, 16:17] + 1e-16
    agg = (sk + jnp.dot(sw[:, 0:16], we_ref[...],
                        preferred_element_type=f32)) / denom
    hd = hd_ref[...]
    out = lax.dot_general(hd, ow_ref[:, 0:D_OUT],
                          (((1,), (1,)), ((), ())),
                          preferred_element_type=f32)
    out = out + lax.dot_general(agg, ow_ref[:, D_OUT:2 * D_OUT],
                                (((1,), (1,)), ((), ())),
                                preferred_element_type=f32)
    out_ref[...] = out + ob_ref[...]


def _final(hd, skp, swp, W_e, out_w, ob):
    nblocks = N // NBLK
    return pl.pallas_call(
        _final_body,
        grid=(nblocks,),
        in_specs=[
            pl.BlockSpec((NBLK, D_OUT), lambda i: (i, 0)),
            pl.BlockSpec((NC, NBLK, DH), lambda i: (0, i, 0)),
            pl.BlockSpec((NC, NBLK, SW_COLS), lambda i: (0, i, 0)),
            pl.BlockSpec((D_EDGE, D_OUT), lambda i: (0, 0)),
            pl.BlockSpec((D_OUT, 2 * D_OUT), lambda i: (0, 0)),
            pl.BlockSpec((1, D_OUT), lambda i: (0, 0)),
        ],
        out_specs=pl.BlockSpec((NBLK, D_OUT), lambda i: (i, 0)),
        out_shape=jax.ShapeDtypeStruct((N, D_OUT), jnp.float32),
    )(hd, skp, swp, W_e, out_w, ob)


def kernel(edge_index, feat_src, feat_dst, edge_weight, W_n, W_e, Q, K,
           att_w, att_b, out_w, out_b):
    src = edge_index[0]
    dst = edge_index[1]
    a1 = att_w[0, 0:D_OUT].reshape(D_OUT, 1)
    a2 = att_w[0, D_OUT:2 * D_OUT].reshape(D_OUT, 1)
    a3 = att_w[0, 2 * D_OUT:3 * D_OUT].reshape(D_OUT, 1)
    ab = att_b.reshape(1, 1)
    ob = out_b.reshape(1, D_OUT)

    hd, klo, khi, ad, asrc = _node_proj(feat_src, feat_dst, W_n, Q, K, a1, a2)
    beta = _beta(edge_weight, W_e, a3, ab)
    skp, swp = _sc_edge(src, dst, beta.reshape(E), ad.reshape(N),
                        asrc.reshape(N), klo, khi, edge_weight)
    return _final(hd, skp, swp, W_e, out_w, ob)


# speedup vs baseline: 7.1903x; 1.3470x over previous
import functools

import jax
import jax.numpy as jnp
from jax import lax
from jax.experimental import pallas as pl
from jax.experimental.pallas import tpu as pltpu
from jax.experimental.pallas import tpu_sc as plsc

N = 10000
E = 320000
D_IN = 128
D_OUT = 128
D_EDGE = 16
DH = D_OUT // 2

NC = 2
NS = 16
EPT = E // NS
CHUNK = 80
NCHUNK = EPT // CHUNK
NROWBLK = N // CHUNK
NBLK = 1000
SW_COLS = 32


def _node_proj_body(fs_ref, fd_ref, wn_ref, q_ref, kw_ref, a1_ref, a2_ref,
                    hd_ref, klo_ref, khi_ref, ad_ref, as_ref):
    f32 = jnp.float32
    hd = jnp.dot(fd_ref[...], wn_ref[...], preferred_element_type=f32)
    hs = jnp.dot(fs_ref[...], wn_ref[...], preferred_element_type=f32)
    kk = jnp.dot(hs, kw_ref[...], preferred_element_type=f32)
    hd_ref[...] = hd
    klo_ref[...] = kk[:, 0:DH]
    khi_ref[...] = kk[:, DH:D_OUT]
    qa1 = jnp.dot(q_ref[...], a1_ref[...], preferred_element_type=f32)
    ad_ref[...] = jnp.dot(hd, qa1, preferred_element_type=f32)
    as_ref[...] = jnp.dot(kk, a2_ref[...], preferred_element_type=f32)


def _node_proj(feat_src, feat_dst, W_n, Q, K, a1, a2):
    nblocks = N // NBLK
    full = lambda shape: pl.BlockSpec(shape, lambda i: (0, 0))
    blk = lambda w: pl.BlockSpec((NBLK, w), lambda i: (i, 0))
    return pl.pallas_call(
        _node_proj_body,
        grid=(nblocks,),
        in_specs=[
            blk(D_IN), blk(D_IN),
            full((D_IN, D_OUT)), full((D_OUT, D_OUT)), full((D_OUT, D_OUT)),
            full((D_OUT, 1)), full((D_OUT, 1)),
        ],
        out_specs=[blk(D_OUT), blk(DH), blk(DH), blk(1), blk(1)],
        out_shape=[
            jax.ShapeDtypeStruct((N, D_OUT), jnp.float32),
            jax.ShapeDtypeStruct((N, DH), jnp.float32),
            jax.ShapeDtypeStruct((N, DH), jnp.float32),
            jax.ShapeDtypeStruct((N, 1), jnp.float32),
            jax.ShapeDtypeStruct((N, 1), jnp.float32),
        ],
    )(feat_src, feat_dst, W_n, Q, K, a1, a2)


def _beta_body(ew_ref, we_ref, a3_ref, ab_ref, beta_ref):
    f32 = jnp.float32
    wv = jnp.dot(we_ref[...], a3_ref[...], preferred_element_type=f32)
    beta_ref[...] = (
        jnp.dot(ew_ref[...], wv, preferred_element_type=f32) + ab_ref[0, 0]
    )


def _beta(edge_weight, W_e, a3, ab):
    eb = 8000
    return pl.pallas_call(
        _beta_body,
        grid=(E // eb,),
        in_specs=[
            pl.BlockSpec((eb, D_EDGE), lambda i: (i, 0)),
            pl.BlockSpec((D_EDGE, D_OUT), lambda i: (0, 0)),
            pl.BlockSpec((D_OUT, 1), lambda i: (0, 0)),
            pl.BlockSpec((1, 1), lambda i: (0, 0)),
        ],
        out_specs=pl.BlockSpec((eb, 1), lambda i: (i, 0)),
        out_shape=jax.ShapeDtypeStruct((E, 1), jnp.float32),
    )(edge_weight, W_e, a3, ab)


def _sc_edge_body(src_hbm, dst_hbm, beta_hbm, ad_hbm, as_hbm, klo_hbm,
                  khi_hbm, ew_hbm,
                  sk_out, sw_out,
                  ad_v, as_v, src0, src1, dst0, dst1, beta0, beta1,
                  ew_v, ews_v, krows0, krows1,
                  sk_sh, sw_sh, sem0, sem1):
    c = lax.axis_index("c")
    s = lax.axis_index("s")
    z16 = jnp.zeros((16,), jnp.float32)
    lane = lax.broadcasted_iota(jnp.int32, (16,), 0)
    srcb, dstb, betab = (src0, src1), (dst0, dst1), (beta0, beta1)
    krowsb, semb = (krows0, krows1), (sem0, sem1)

    pltpu.sync_copy(ad_hbm, ad_v)
    pltpu.sync_copy(as_hbm, as_v)

    def zero_body(j, _):
        for v in range(DH // 16):
            krows0[j, pl.ds(v * 16, 16)] = z16
        for v in range(SW_COLS // 16):
            ews_v[j, pl.ds(v * 16, 16)] = z16
        return 0

    lax.fori_loop(0, CHUNK, zero_body, 0)
    for t in range(pl.cdiv(NROWBLK, NS)):
        bid = s + NS * t
        @pl.when(bid < NROWBLK)
        def _():
            r0 = bid * CHUNK
            pltpu.sync_copy(krows0, sk_sh.at[pl.ds(r0, CHUNK)])
            pltpu.sync_copy(ews_v, sw_sh.at[pl.ds(r0, CHUNK)])
    plsc.subcore_barrier()

    def load_scalars(idx, par):
        base = s * EPT + idx * CHUNK
        pltpu.sync_copy(src_hbm.at[pl.ds(base, CHUNK)], srcb[par])
        pltpu.sync_copy(dst_hbm.at[pl.ds(base, CHUNK)], dstb[par])
        pltpu.sync_copy(beta_hbm.at[pl.ds(base, CHUNK)], betab[par])

    def start_gather(par):
        @pl.when(c == 0)
        def _():
            pltpu.async_copy(klo_hbm.at[srcb[par]], krowsb[par], semb[par])

        @pl.when(c == 1)
        def _():
            pltpu.async_copy(khi_hbm.at[srcb[par]], krowsb[par], semb[par])

    def process(idx, par):
        exs = []
        for g in range(CHUNK // 16):
            di = dstb[par][pl.ds(g * 16, 16)]
            si = srcb[par][pl.ds(g * 16, 16)]
            z = (plsc.load_gather(ad_v, [di]) + plsc.load_gather(as_v, [si])
                 + betab[par][pl.ds(g * 16, 16)])
            lg = jnp.where(z >= 0.0, z, 0.2 * z)
            exs.append(jnp.exp(lg))

        do_sw = jnp.logical_or(
            jnp.logical_and(c == 0, idx < NCHUNK // 2),
            jnp.logical_and(c == 1, idx >= NCHUNK // 2))

        @pl.when(do_sw)
        def _():
            base = s * EPT + idx * CHUNK
            pltpu.sync_copy(ew_hbm.at[pl.ds(base, CHUNK)], ew_v)

        pltpu.make_async_copy(klo_hbm.at[srcb[par]], krowsb[par],
                              semb[par]).wait()
        kr = krowsb[par]
        for g in range(CHUNK // 16):
            ex16 = exs[g]
            for j in range(16):
                row = g * 16 + j
                bex = ex16.at[jnp.full((16,), j, jnp.int32)].get(
                    mode="promise_in_bounds")
                for v in range(DH // 16):
                    kr[row, pl.ds(v * 16, 16)] = (
                        kr[row, pl.ds(v * 16, 16)] * bex)
                ews_v[row, pl.ds(0, 16)] = ew_v[row, pl.ds(0, 16)] * bex
                ews_v[row, pl.ds(16, 16)] = jnp.where(lane == 0, bex, 0.0)
        pltpu.sync_copy(kr, sk_sh.at[dstb[par]], add=True)

        @pl.when(do_sw)
        def _():
            pltpu.sync_copy(ews_v, sw_sh.at[dstb[par]], add=True)

    load_scalars(0, 0)
    start_gather(0)

    def loop_body(t, _):
        i0 = 2 * t
        load_scalars(i0 + 1, 1)
        start_gather(1)
        process(i0, 0)

        @pl.when(i0 + 2 < NCHUNK)
        def _():
            load_scalars(i0 + 2, 0)
            start_gather(0)

        process(i0 + 1, 1)
        return 0

    lax.fori_loop(0, NCHUNK // 2, loop_body, 0)
    plsc.subcore_barrier()

    for t in range(pl.cdiv(NROWBLK, NS)):
        bid = s + NS * t
        @pl.when(bid < NROWBLK)
        def _():
            r0 = bid * CHUNK
            pltpu.sync_copy(sk_sh.at[pl.ds(r0, CHUNK)],
                            sk_out.at[c, pl.ds(r0, CHUNK)])
            pltpu.sync_copy(sw_sh.at[pl.ds(r0, CHUNK)],
                            sw_out.at[c, pl.ds(r0, CHUNK)])


def _sc_edge(src, dst, beta, ad, asrc, klo, khi, edge_weight):
    mesh = plsc.VectorSubcoreMesh(
        core_axis_name="c", subcore_axis_name="s",
        num_cores=NC, num_subcores=NS)
    f32 = jnp.float32
    return pl.kernel(
        _sc_edge_body,
        out_type=[
            jax.ShapeDtypeStruct((NC, N, DH), f32),
            jax.ShapeDtypeStruct((NC, N, SW_COLS), f32),
        ],
        mesh=mesh,
        compiler_params=pltpu.CompilerParams(
            needs_layout_passes=False, use_tc_tiling_on_sc=False),
        scratch_types=[
            pltpu.VMEM((N,), f32),
            pltpu.VMEM((N,), f32),
            pltpu.VMEM((CHUNK,), jnp.int32),
            pltpu.VMEM((CHUNK,), jnp.int32),
            pltpu.VMEM((CHUNK,), jnp.int32),
            pltpu.VMEM((CHUNK,), jnp.int32),
            pltpu.VMEM((CHUNK,), f32),
            pltpu.VMEM((CHUNK,), f32),
            pltpu.VMEM((CHUNK, D_EDGE), f32),
            pltpu.VMEM((CHUNK, SW_COLS), f32),
            pltpu.VMEM((CHUNK, DH), f32),
            pltpu.VMEM((CHUNK, DH), f32),
            pltpu.VMEM_SHARED((N, DH), f32),
            pltpu.VMEM_SHARED((N, SW_COLS), f32),
            pltpu.SemaphoreType.DMA,
            pltpu.SemaphoreType.DMA,
        ],
    )(src, dst, beta, ad, asrc, klo, khi, edge_weight)


def _final_body(hd_ref, skp_ref, swp_ref, we_ref, ow_ref, ob_ref, out_ref):
    f32 = jnp.float32
    sk = jnp.concatenate([skp_ref[0], skp_ref[1]], axis=1)
    sw = swp_ref[0] + swp_ref[1]
    denom = sw[:, 16:17] + 1e-16
    agg = (sk + jnp.dot(sw[:, 0:16], we_ref[...],
                        preferred_element_type=f32)) / denom
    hd = hd_ref[...]
    out = lax.dot_general(hd, ow_ref[:, 0:D_OUT],
                          (((1,), (1,)), ((), ())),
                          preferred_element_type=f32)
    out = out + lax.dot_general(agg, ow_ref[:, D_OUT:2 * D_OUT],
                                (((1,), (1,)), ((), ())),
                                preferred_element_type=f32)
    out_ref[...] = out + ob_ref[...]


def _final(hd, skp, swp, W_e, out_w, ob):
    nblocks = N // NBLK
    return pl.pallas_call(
        _final_body,
        grid=(nblocks,),
        in_specs=[
            pl.BlockSpec((NBLK, D_OUT), lambda i: (i, 0)),
            pl.BlockSpec((NC, NBLK, DH), lambda i: (0, i, 0)),
            pl.BlockSpec((NC, NBLK, SW_COLS), lambda i: (0, i, 0)),
            pl.BlockSpec((D_EDGE, D_OUT), lambda i: (0, 0)),
            pl.BlockSpec((D_OUT, 2 * D_OUT), lambda i: (0, 0)),
            pl.BlockSpec((1, D_OUT), lambda i: (0, 0)),
        ],
        out_specs=pl.BlockSpec((NBLK, D_OUT), lambda i: (i, 0)),
        out_shape=jax.ShapeDtypeStruct((N, D_OUT), jnp.float32),
    )(hd, skp, swp, W_e, out_w, ob)


def kernel(edge_index, feat_src, feat_dst, edge_weight, W_n, W_e, Q, K,
           att_w, att_b, out_w, out_b):
    src = edge_index[0]
    dst = edge_index[1]
    a1 = att_w[0, 0:D_OUT].reshape(D_OUT, 1)
    a2 = att_w[0, D_OUT:2 * D_OUT].reshape(D_OUT, 1)
    a3 = att_w[0, 2 * D_OUT:3 * D_OUT].reshape(D_OUT, 1)
    ab = att_b.reshape(1, 1)
    ob = out_b.reshape(1, D_OUT)

    hd, klo, khi, ad, asrc = _node_proj(feat_src, feat_dst, W_n, Q, K, a1, a2)
    beta = _beta(edge_weight, W_e, a3, ab)
    skp, swp = _sc_edge(src, dst, beta.reshape(E), ad.reshape(N),
                        asrc.reshape(N), klo, khi, edge_weight)
    return _final(hd, skp, swp, W_e, out_w, ob)


# speedup vs baseline: 7.9714x; 1.1086x over previous
import functools

import jax
import jax.numpy as jnp
from jax import lax
from jax.experimental import pallas as pl
from jax.experimental.pallas import tpu as pltpu
from jax.experimental.pallas import tpu_sc as plsc

N = 10000
E = 320000
D_IN = 128
D_OUT = 128
D_EDGE = 16
DH = D_OUT // 2

NC = 2
NS = 16
EPT = E // NS
CHUNK = 80
NCHUNK = EPT // CHUNK
NROWBLK = N // CHUNK
NBLK = 1000
SW_COLS = 32


def _node_proj_body(fs_ref, fd_ref, wn_ref, q_ref, kw_ref, a1_ref, a2_ref,
                    we_ref, a3_ref, ab_ref,
                    hd_ref, klo_ref, khi_ref, ad_ref, as_ref, wv_ref):
    f32 = jnp.float32
    hd = jnp.dot(fd_ref[...], wn_ref[...], preferred_element_type=f32)
    hs = jnp.dot(fs_ref[...], wn_ref[...], preferred_element_type=f32)
    kk = jnp.dot(hs, kw_ref[...], preferred_element_type=f32)
    hd_ref[...] = hd
    klo_ref[...] = kk[:, 0:DH]
    khi_ref[...] = kk[:, DH:D_OUT]
    qa1 = jnp.dot(q_ref[...], a1_ref[...], preferred_element_type=f32)
    ad_ref[...] = jnp.dot(hd, qa1, preferred_element_type=f32) + ab_ref[0, 0]
    as_ref[...] = jnp.dot(kk, a2_ref[...], preferred_element_type=f32)
    wv_ref[...] = jnp.dot(we_ref[...], a3_ref[...], preferred_element_type=f32)


def _node_proj(feat_src, feat_dst, W_n, Q, K, a1, a2, W_e, a3, ab):
    nblocks = N // NBLK
    full = lambda shape: pl.BlockSpec(shape, lambda i: (0, 0))
    blk = lambda w: pl.BlockSpec((NBLK, w), lambda i: (i, 0))
    return pl.pallas_call(
        _node_proj_body,
        grid=(nblocks,),
        in_specs=[
            blk(D_IN), blk(D_IN),
            full((D_IN, D_OUT)), full((D_OUT, D_OUT)), full((D_OUT, D_OUT)),
            full((D_OUT, 1)), full((D_OUT, 1)),
            full((D_EDGE, D_OUT)), full((D_OUT, 1)), full((1, 1)),
        ],
        out_specs=[blk(D_OUT), blk(DH), blk(DH), blk(1), blk(1),
                   full((D_EDGE, 1))],
        out_shape=[
            jax.ShapeDtypeStruct((N, D_OUT), jnp.float32),
            jax.ShapeDtypeStruct((N, DH), jnp.float32),
            jax.ShapeDtypeStruct((N, DH), jnp.float32),
            jax.ShapeDtypeStruct((N, 1), jnp.float32),
            jax.ShapeDtypeStruct((N, 1), jnp.float32),
            jax.ShapeDtypeStruct((D_EDGE, 1), jnp.float32),
        ],
    )(feat_src, feat_dst, W_n, Q, K, a1, a2, W_e, a3, ab)


def _sc_edge_body(src_hbm, dst_hbm, ad_hbm, as_hbm, klo_hbm,
                  khi_hbm, ewt_hbm, wv_hbm,
                  sk_out, sw_out,
                  ad_v, as_v, wv_v, src0, src1, dst0, dst1,
                  ewt0, ewt1, ews_v, krows0, krows1,
                  sk_sh, sw_sh, sem0, sem1):
    c = lax.axis_index("c")
    s = lax.axis_index("s")
    z16 = jnp.zeros((16,), jnp.float32)
    lane = lax.broadcasted_iota(jnp.int32, (16,), 0)
    srcb, dstb = (src0, src1), (dst0, dst1)
    ewtb = (ewt0, ewt1)
    krowsb, semb = (krows0, krows1), (sem0, sem1)

    pltpu.sync_copy(ad_hbm, ad_v)
    pltpu.sync_copy(as_hbm, as_v)
    pltpu.sync_copy(wv_hbm, wv_v)
    wv16 = wv_v[...]
    wvb = [wv16.at[jnp.full((16,), v, jnp.int32)].get(mode="promise_in_bounds")
           for v in range(D_EDGE)]

    def zero_body(j, _):
        for v in range(DH // 16):
            krows0[j, pl.ds(v * 16, 16)] = z16
        for v in range(SW_COLS // 16):
            ews_v[j, pl.ds(v * 16, 16)] = z16
        return 0

    lax.fori_loop(0, CHUNK, zero_body, 0)
    for t in range(pl.cdiv(NROWBLK, NS)):
        bid = s + NS * t
        @pl.when(bid < NROWBLK)
        def _():
            r0 = bid * CHUNK
            pltpu.sync_copy(krows0, sk_sh.at[pl.ds(r0, CHUNK)])
            pltpu.sync_copy(ews_v, sw_sh.at[pl.ds(r0, CHUNK)])
    plsc.subcore_barrier()

    def load_scalars(idx, par):
        base = s * EPT + idx * CHUNK
        pltpu.sync_copy(src_hbm.at[pl.ds(base, CHUNK)], srcb[par])
        pltpu.sync_copy(dst_hbm.at[pl.ds(base, CHUNK)], dstb[par])
        pltpu.sync_copy(ewt_hbm.at[s * NCHUNK + idx], ewtb[par])

    def start_gather(par):
        @pl.when(c == 0)
        def _():
            pltpu.async_copy(klo_hbm.at[srcb[par]], krowsb[par], semb[par])

        @pl.when(c == 1)
        def _():
            pltpu.async_copy(khi_hbm.at[srcb[par]], krowsb[par], semb[par])

    def process(idx, par):
        ewt = ewtb[par]
        exs = []
        for g in range(CHUNK // 16):
            acc = wvb[0] * ewt[0, pl.ds(g * 16, 16)]
            for v in range(1, D_EDGE):
                acc = acc + wvb[v] * ewt[v, pl.ds(g * 16, 16)]
            di = dstb[par][pl.ds(g * 16, 16)]
            si = srcb[par][pl.ds(g * 16, 16)]
            z = (plsc.load_gather(ad_v, [di]) + plsc.load_gather(as_v, [si])
                 + acc)
            lg = jnp.where(z >= 0.0, z, 0.2 * z)
            exs.append(jnp.exp(lg))

        do_sw = jnp.logical_or(
            jnp.logical_and(c == 0, idx < NCHUNK // 2),
            jnp.logical_and(c == 1, idx >= NCHUNK // 2))

        pltpu.make_async_copy(klo_hbm.at[srcb[par]], krowsb[par],
                              semb[par]).wait()
        kr = krowsb[par]
        for g in range(CHUNK // 16):
            ex16 = exs[g]
            for j in range(16):
                row = g * 16 + j
                bex = ex16.at[jnp.full((16,), j, jnp.int32)].get(
                    mode="promise_in_bounds")
                for v in range(DH // 16):
                    kr[row, pl.ds(v * 16, 16)] = (
                        kr[row, pl.ds(v * 16, 16)] * bex)
                ewrow = plsc.load_gather(
                    ewt, [lane, jnp.full((16,), row, jnp.int32)])
                ews_v[row, pl.ds(0, 16)] = ewrow * bex
                ews_v[row, pl.ds(16, 16)] = jnp.where(lane == 0, bex, 0.0)
        pltpu.sync_copy(kr, sk_sh.at[dstb[par]], add=True)

        @pl.when(do_sw)
        def _():
            pltpu.sync_copy(ews_v, sw_sh.at[dstb[par]], add=True)

    load_scalars(0, 0)
    start_gather(0)

    def loop_body(t, _):
        i0 = 2 * t
        load_scalars(i0 + 1, 1)
        start_gather(1)
        process(i0, 0)

        @pl.when(i0 + 2 < NCHUNK)
        def _():
            load_scalars(i0 + 2, 0)
            start_gather(0)

        process(i0 + 1, 1)
        return 0

    lax.fori_loop(0, NCHUNK // 2, loop_body, 0)
    plsc.subcore_barrier()

    for t in range(pl.cdiv(NROWBLK, NS)):
        bid = s + NS * t
        @pl.when(bid < NROWBLK)
        def _():
            r0 = bid * CHUNK
            pltpu.sync_copy(sk_sh.at[pl.ds(r0, CHUNK)],
                            sk_out.at[c, pl.ds(r0, CHUNK)])
            pltpu.sync_copy(sw_sh.at[pl.ds(r0, CHUNK)],
                            sw_out.at[c, pl.ds(r0, CHUNK)])


def _sc_edge(src, dst, ad, asrc, klo, khi, ewt, wv):
    mesh = plsc.VectorSubcoreMesh(
        core_axis_name="c", subcore_axis_name="s",
        num_cores=NC, num_subcores=NS)
    f32 = jnp.float32
    return pl.kernel(
        _sc_edge_body,
        out_type=[
            jax.ShapeDtypeStruct((NC, N, DH), f32),
            jax.ShapeDtypeStruct((NC, N, SW_COLS), f32),
        ],
        mesh=mesh,
        compiler_params=pltpu.CompilerParams(
            needs_layout_passes=False, use_tc_tiling_on_sc=False),
        scratch_types=[
            pltpu.VMEM((N,), f32),
            pltpu.VMEM((N,), f32),
            pltpu.VMEM((D_EDGE,), f32),
            pltpu.VMEM((CHUNK,), jnp.int32),
            pltpu.VMEM((CHUNK,), jnp.int32),
            pltpu.VMEM((CHUNK,), jnp.int32),
            pltpu.VMEM((CHUNK,), jnp.int32),
            pltpu.VMEM((D_EDGE, CHUNK), f32),
            pltpu.VMEM((D_EDGE, CHUNK), f32),
            pltpu.VMEM((CHUNK, SW_COLS), f32),
            pltpu.VMEM((CHUNK, DH), f32),
            pltpu.VMEM((CHUNK, DH), f32),
            pltpu.VMEM_SHARED((N, DH), f32),
            pltpu.VMEM_SHARED((N, SW_COLS), f32),
            pltpu.SemaphoreType.DMA,
            pltpu.SemaphoreType.DMA,
        ],
    )(src, dst, ad, asrc, klo, khi, ewt, wv)


def _final_body(hd_ref, skp_ref, swp_ref, we_ref, ow_ref, ob_ref, out_ref):
    f32 = jnp.float32
    sk = jnp.concatenate([skp_ref[0], skp_ref[1]], axis=1)
    sw = swp_ref[0] + swp_ref[1]
    denom = sw[:, 16:17] + 1e-16
    agg = (sk + jnp.dot(sw[:, 0:16], we_ref[...],
                        preferred_element_type=f32)) / denom
    hd = hd_ref[...]
    out = lax.dot_general(hd, ow_ref[:, 0:D_OUT],
                          (((1,), (1,)), ((), ())),
                          preferred_element_type=f32)
    out = out + lax.dot_general(agg, ow_ref[:, D_OUT:2 * D_OUT],
                                (((1,), (1,)), ((), ())),
                                preferred_element_type=f32)
    out_ref[...] = out + ob_ref[...]


def _final(hd, skp, swp, W_e, out_w, ob):
    nblocks = N // NBLK
    return pl.pallas_call(
        _final_body,
        grid=(nblocks,),
        in_specs=[
            pl.BlockSpec((NBLK, D_OUT), lambda i: (i, 0)),
            pl.BlockSpec((NC, NBLK, DH), lambda i: (0, i, 0)),
            pl.BlockSpec((NC, NBLK, SW_COLS), lambda i: (0, i, 0)),
            pl.BlockSpec((D_EDGE, D_OUT), lambda i: (0, 0)),
            pl.BlockSpec((D_OUT, 2 * D_OUT), lambda i: (0, 0)),
            pl.BlockSpec((1, D_OUT), lambda i: (0, 0)),
        ],
        out_specs=pl.BlockSpec((NBLK, D_OUT), lambda i: (i, 0)),
        out_shape=jax.ShapeDtypeStruct((N, D_OUT), jnp.float32),
    )(hd, skp, swp, W_e, out_w, ob)


def kernel(edge_index, feat_src, feat_dst, edge_weight, W_n, W_e, Q, K,
           att_w, att_b, out_w, out_b):
    src = edge_index[0]
    dst = edge_index[1]
    a1 = att_w[0, 0:D_OUT].reshape(D_OUT, 1)
    a2 = att_w[0, D_OUT:2 * D_OUT].reshape(D_OUT, 1)
    a3 = att_w[0, 2 * D_OUT:3 * D_OUT].reshape(D_OUT, 1)
    ab = att_b.reshape(1, 1)
    ob = out_b.reshape(1, D_OUT)

    hd, klo, khi, ad, asrc, wv = _node_proj(
        feat_src, feat_dst, W_n, Q, K, a1, a2, W_e, a3, ab)
    ewt_blocks = jnp.transpose(
        edge_weight.reshape(E // CHUNK, CHUNK, D_EDGE), (0, 2, 1))
    skp, swp = _sc_edge(src, dst, ad.reshape(N), asrc.reshape(N),
                        klo, khi, ewt_blocks, wv.reshape(D_EDGE))
    return _final(hd, skp, swp, W_e, out_w, ob)
